# baseline (speedup 1.0000x reference)
import functools

import jax
import jax.numpy as jnp
from jax import lax
from jax.experimental import pallas as pl
from jax.experimental.pallas import tpu as pltpu
from jax.experimental.pallas import tpu_sc as plsc

_D = 128
_ROWS = 1000
_B = 16384
_NC = 2
_NS = 16
_NW = _NC * _NS
_BPW = _B // _NW
_L = 16
_NCH = 2
_CH = _BPW // _NCH
_NSTAGE = 5


def _sc_gather(t, pe):
    mesh = plsc.VectorSubcoreMesh(core_axis_name="c", subcore_axis_name="s")

    @functools.partial(
        pl.kernel,
        mesh=mesh,
        out_type=jax.ShapeDtypeStruct((_B, _D), jnp.float32),
        scratch_types=[
            pltpu.VMEM((_BPW,), jnp.int32),
            pltpu.VMEM((_BPW, _D), jnp.float32),
            pltpu.VMEM_SHARED((_ROWS, _D), jnp.float32),
        ] + [pltpu.SemaphoreType.DMA] * (_NCH + 1),
    )
    def k(idx_hbm, table_hbm, out_hbm, idx_v, rows_v, table_s, *sems):
        gsems, wsem = sems[:_NCH], sems[_NCH]
        sid = lax.axis_index("s")
        wid = sid * _NC + lax.axis_index("c")
        base = wid * _BPW

        rpt = _ROWS // _NSTAGE
        @pl.when(sid < _NSTAGE)
        def _():
            pltpu.sync_copy(table_hbm.at[pl.ds(sid * rpt, rpt)],
                            table_s.at[pl.ds(sid * rpt, rpt)])

        pltpu.sync_copy(idx_hbm.at[pl.ds(base, _BPW)], idx_v)

        def clip_body(i, carry):
            off = pl.multiple_of(i * _L, _L)
            v = idx_v[pl.ds(off, _L)]
            idx_v[pl.ds(off, _L)] = jnp.minimum(
                jnp.maximum(v, 0), _ROWS - 1)
            return carry
        lax.fori_loop(0, _BPW // _L, clip_body, 0)
        plsc.subcore_barrier()

        gathers = []
        for c in range(_NCH):
            gathers.append(pltpu.async_copy(
                table_s.at[idx_v.at[pl.ds(c * _CH, _CH)]],
                rows_v.at[pl.ds(c * _CH, _CH)],
                gsems[c]))

        writes = []
        for c in range(_NCH):
            gathers[c].wait()
            writes.append(pltpu.async_copy(
                rows_v.at[pl.ds(c * _CH, _CH)],
                out_hbm.at[pl.ds(base + c * _CH, _CH)],
                wsem))
        for w in writes:
            w.wait()

    return k(t, pe)


def kernel(t, pe):
    return _sc_gather(t.astype(jnp.int32), pe)


# speedup vs baseline: 1.0546x; 1.0546x over previous
import functools

import jax
import jax.numpy as jnp
from jax import lax
from jax.experimental import pallas as pl
from jax.experimental.pallas import tpu as pltpu
from jax.experimental.pallas import tpu_sc as plsc

_D = 128
_ROWS = 1000
_B = 16384
_NC = 2
_NS = 16
_NW = _NC * _NS
_BPW = _B // _NW
_L = 16
_NCH = 16
_CH = _BPW // _NCH
_NSTAGE = 5


def _sc_gather(t, pe):
    mesh = plsc.VectorSubcoreMesh(core_axis_name="c", subcore_axis_name="s")

    @functools.partial(
        pl.kernel,
        mesh=mesh,
        out_type=jax.ShapeDtypeStruct((_B, _D), jnp.float32),
        scratch_types=[
            pltpu.VMEM((_BPW,), jnp.int32),
            pltpu.VMEM((_BPW, _D), jnp.float32),
            pltpu.VMEM_SHARED((_ROWS, _D), jnp.float32),
        ] + [pltpu.SemaphoreType.DMA] * (_NCH + 1),
    )
    def k(idx_hbm, table_hbm, out_hbm, idx_v, rows_v, table_s, *sems):
        gsems, wsem = sems[:_NCH], sems[_NCH]
        sid = lax.axis_index("s")
        wid = sid * _NC + lax.axis_index("c")
        base = wid * _BPW

        rpt = _ROWS // _NSTAGE
        @pl.when(sid < _NSTAGE)
        def _():
            pltpu.sync_copy(table_hbm.at[pl.ds(sid * rpt, rpt)],
                            table_s.at[pl.ds(sid * rpt, rpt)])

        pltpu.sync_copy(idx_hbm.at[pl.ds(base, _BPW)], idx_v)

        def clip_body(i, carry):
            off = pl.multiple_of(i * _L, _L)
            v = idx_v[pl.ds(off, _L)]
            idx_v[pl.ds(off, _L)] = jnp.minimum(
                jnp.maximum(v, 0), _ROWS - 1)
            return carry
        lax.fori_loop(0, _BPW // _L, clip_body, 0)
        plsc.subcore_barrier()

        gathers = []
        for c in range(_NCH):
            gathers.append(pltpu.async_copy(
                table_s.at[idx_v.at[pl.ds(c * _CH, _CH)]],
                rows_v.at[pl.ds(c * _CH, _CH)],
                gsems[c]))

        writes = []
        for c in range(_NCH):
            gathers[c].wait()
            writes.append(pltpu.async_copy(
                rows_v.at[pl.ds(c * _CH, _CH)],
                out_hbm.at[pl.ds(base + c * _CH, _CH)],
                wsem))
        for w in writes:
            w.wait()

    return k(t, pe)


def kernel(t, pe):
    return _sc_gather(t.astype(jnp.int32), pe)


# speedup vs baseline: 1.0870x; 1.0308x over previous
import functools

import jax
import jax.numpy as jnp
from jax import lax
from jax.experimental import pallas as pl
from jax.experimental.pallas import tpu as pltpu
from jax.experimental.pallas import tpu_sc as plsc

_D = 128
_ROWS = 1000
_B = 16384
_NC = 2
_NS = 16
_NW = _NC * _NS
_BPW = _B // _NW
_L = 16
_NCH = 8
_CH = _BPW // _NCH
_NSTAGE = 5


def _sc_gather(t, pe):
    mesh = plsc.VectorSubcoreMesh(core_axis_name="c", subcore_axis_name="s")

    @functools.partial(
        pl.kernel,
        mesh=mesh,
        out_type=jax.ShapeDtypeStruct((_B, _D), jnp.float32),
        scratch_types=[
            pltpu.VMEM((_BPW,), jnp.int32),
            pltpu.VMEM((_BPW, _D), jnp.float32),
            pltpu.VMEM_SHARED((_ROWS, _D), jnp.float32),
        ] + [pltpu.SemaphoreType.DMA] * (_NCH + 3),
    )
    def k(idx_hbm, table_hbm, out_hbm, idx_v, rows_v, table_s, *sems):
        gsems, wsem = sems[:_NCH], sems[_NCH]
        isem, ssem = sems[_NCH + 1], sems[_NCH + 2]
        sid = lax.axis_index("s")
        wid = sid * _NC + lax.axis_index("c")
        base = wid * _BPW

        icp = pltpu.async_copy(idx_hbm.at[pl.ds(base, _BPW)], idx_v, isem)
        rpt = _ROWS // _NSTAGE
        @pl.when(sid < _NSTAGE)
        def _():
            pltpu.async_copy(table_hbm.at[pl.ds(sid * rpt, rpt)],
                             table_s.at[pl.ds(sid * rpt, rpt)], ssem)

        icp.wait()

        def clip_body(i, carry):
            off = pl.multiple_of(i * _L, _L)
            v = idx_v[pl.ds(off, _L)]
            idx_v[pl.ds(off, _L)] = jnp.minimum(
                jnp.maximum(v, 0), _ROWS - 1)
            return carry
        lax.fori_loop(0, _BPW // _L, clip_body, 0)

        @pl.when(sid < _NSTAGE)
        def _():
            pltpu.make_async_copy(
                table_hbm.at[pl.ds(sid * rpt, rpt)],
                table_s.at[pl.ds(sid * rpt, rpt)], ssem).wait()
        plsc.subcore_barrier()

        gathers = []
        for c in range(_NCH):
            gathers.append(pltpu.async_copy(
                table_s.at[idx_v.at[pl.ds(c * _CH, _CH)]],
                rows_v.at[pl.ds(c * _CH, _CH)],
                gsems[c]))

        writes = []
        for c in range(_NCH):
            gathers[c].wait()
            writes.append(pltpu.async_copy(
                rows_v.at[pl.ds(c * _CH, _CH)],
                out_hbm.at[pl.ds(base + c * _CH, _CH)],
                wsem))
        for w in writes:
            w.wait()

    return k(t, pe)


def kernel(t, pe):
    return _sc_gather(t.astype(jnp.int32), pe)


# speedup vs baseline: 1.0872x; 1.0001x over previous
import functools

import jax
import jax.numpy as jnp
from jax import lax
from jax.experimental import pallas as pl
from jax.experimental.pallas import tpu as pltpu
from jax.experimental.pallas import tpu_sc as plsc

_D = 128
_ROWS = 1000
_B = 16384
_NC = 2
_NS = 16
_NW = _NC * _NS
_BPW = _B // _NW
_L = 16
_NCH = 8
_CH = _BPW // _NCH
_NSTAGE = 5


def _sc_gather(t, pe):
    mesh = plsc.VectorSubcoreMesh(core_axis_name="c", subcore_axis_name="s")

    @functools.partial(
        pl.kernel,
        mesh=mesh,
        out_type=jax.ShapeDtypeStruct((_B, _D), jnp.float32),
        scratch_types=[
            pltpu.VMEM((_BPW,), jnp.int32),
            pltpu.VMEM((_BPW, _D), jnp.float32),
            pltpu.VMEM_SHARED((_ROWS, _D), jnp.float32),
        ] + [pltpu.SemaphoreType.DMA] * (_NCH + 3),
    )
    def k(idx_hbm, table_hbm, out_hbm, idx_v, rows_v, table_s, *sems):
        gsems, wsem = sems[:_NCH], sems[_NCH]
        isem, ssem = sems[_NCH + 1], sems[_NCH + 2]
        sid = lax.axis_index("s")
        wid = sid * _NC + lax.axis_index("c")
        base = wid * _BPW

        icp = pltpu.async_copy(idx_hbm.at[pl.ds(base, _BPW)], idx_v, isem)
        rpt = _ROWS // _NSTAGE
        @pl.when(sid < _NSTAGE)
        def _():
            pltpu.async_copy(table_hbm.at[pl.ds(sid * rpt, rpt)],
                             table_s.at[pl.ds(sid * rpt, rpt)], ssem)

        icp.wait()

        def clip_body(i, carry):
            off = pl.multiple_of(i * _L, _L)
            v = idx_v[pl.ds(off, _L)]
            idx_v[pl.ds(off, _L)] = jnp.minimum(
                jnp.maximum(v, 0), _ROWS - 1)
            return carry
        lax.fori_loop(0, _BPW // _L, clip_body, 0)

        @pl.when(sid < _NSTAGE)
        def _():
            pltpu.make_async_copy(
                table_hbm.at[pl.ds(sid * rpt, rpt)],
                table_s.at[pl.ds(sid * rpt, rpt)], ssem).wait()
        plsc.subcore_barrier()

        gathers = []
        for c in range(_NCH):
            gathers.append(pltpu.async_copy(
                table_s.at[idx_v.at[pl.ds(c * _CH, _CH)]],
                rows_v.at[pl.ds(c * _CH, _CH)],
                gsems[c]))

        writes = []
        for c in range(_NCH):
            gathers[c].wait()
            writes.append(pltpu.async_copy(
                rows_v.at[pl.ds(c * _CH, _CH)],
                out_hbm.at[pl.ds(base + c * _CH, _CH)],
                wsem))
        for w in writes:
            w.wait()

    return k(t, pe)


def kernel(t, pe):
    assert t.shape == (_B,) and pe.shape == (_ROWS, _D), (t.shape, pe.shape)
    return _sc_gather(t.astype(jnp.int32), pe)
